# baseline (speedup 1.0000x reference)
import functools

import jax
import jax.numpy as jnp
from jax import lax
from jax.experimental import pallas as pl
from jax.experimental.pallas import tpu as pltpu
from jax.experimental.pallas import tpu_sc as plsc

_B = 16
_N = 32768
_NC = 2
_NS = 16
_L = 16
_NW = _NC * _NS
_C = _N // _NW
_V = _C // _L

_mesh = plsc.VectorSubcoreMesh(core_axis_name="c", subcore_axis_name="s")


@functools.partial(
    pl.kernel,
    mesh=_mesh,
    out_type=jax.ShapeDtypeStruct((_NW * _B,), jnp.float32),
    scratch_types=[
        pltpu.VMEM((_C,), jnp.float32),
        pltpu.VMEM((_C,), jnp.float32),
        pltpu.VMEM((_C,), jnp.int32),
        pltpu.VMEM((_B,), jnp.float32),
        pltpu.SemaphoreType.DMA,
    ],
)
def _segsum_sc(scores_hbm, gating_hbm, ids_hbm, out_hbm,
               s_v, g_v, i_v, part_v, sem):
    cid = lax.axis_index("c")
    sid = lax.axis_index("s")
    wid = sid * _NC + cid
    base = wid * _C

    c1 = pltpu.async_copy(scores_hbm.at[pl.ds(base, _C)], s_v, sem)
    c2 = pltpu.async_copy(gating_hbm.at[pl.ds(base, _C)], g_v, sem)
    c3 = pltpu.async_copy(ids_hbm.at[pl.ds(base, _C)], i_v, sem)
    c1.wait()
    c2.wait()
    c3.wait()

    first = i_v[pl.ds(0, _L)][0]
    last = i_v[pl.ds(_C - _L, _L)][_L - 1]
    lanes = lax.iota(jnp.int32, _L)

    def lane_sum(acc):
        half = [acc[2 * l] + acc[2 * l + 1] for l in range(_L // 2)]
        while len(half) > 1:
            half = [half[2 * l] + half[2 * l + 1]
                    for l in range(len(half) // 2)]
        return half[0]

    def seg_body(b, part):
        def abody(j, accs):
            new = []
            for k in range(8):
                sl = pl.ds((8 * j + k) * _L, _L)
                new.append(accs[k]
                           + jnp.where(i_v[sl] == b, s_v[sl] * g_v[sl], 0.0))
            return tuple(new)

        z = jnp.zeros((_L,), jnp.float32)
        accs = lax.fori_loop(0, _V // 8, abody, (z,) * 8)
        acc = ((accs[0] + accs[1]) + (accs[2] + accs[3])) + (
            (accs[4] + accs[5]) + (accs[6] + accs[7]))
        return jnp.where(lanes == b, lane_sum(acc), part)

    part_v[...] = lax.fori_loop(first, last + 1, seg_body,
                                jnp.zeros((_L,), jnp.float32))

    pltpu.sync_copy(part_v, out_hbm.at[pl.ds(wid * _B, _B)])


def kernel(scores, gating, segment_ids):
    partials = _segsum_sc(scores, gating, segment_ids.astype(jnp.int32))
    return jnp.sum(partials.reshape(_NW, _B), axis=0)


# speedup vs baseline: 1.0156x; 1.0156x over previous
import functools

import jax
import jax.numpy as jnp
from jax import lax
from jax.experimental import pallas as pl
from jax.experimental.pallas import tpu as pltpu
from jax.experimental.pallas import tpu_sc as plsc

_B = 16
_N = 32768
_NC = 2
_NS = 16
_L = 16
_NW = _NC * _NS
_C = _N // _NW
_V = _C // _L

_mesh = plsc.VectorSubcoreMesh(core_axis_name="c", subcore_axis_name="s")


@functools.partial(
    pl.kernel,
    mesh=_mesh,
    out_type=jax.ShapeDtypeStruct((_NW * _B,), jnp.float32),
    scratch_types=[
        pltpu.VMEM((_C,), jnp.float32),
        pltpu.VMEM((_C,), jnp.float32),
        pltpu.VMEM((_C,), jnp.int32),
        pltpu.VMEM((_B,), jnp.float32),
        pltpu.SemaphoreType.DMA,
    ],
)
def _segsum_sc(scores_hbm, gating_hbm, ids_hbm, out_hbm,
               s_v, g_v, i_v, part_v, sem):
    cid = lax.axis_index("c")
    sid = lax.axis_index("s")
    wid = sid * _NC + cid
    base = wid * _C

    c1 = pltpu.async_copy(scores_hbm.at[pl.ds(base, _C)], s_v, sem)
    c2 = pltpu.async_copy(gating_hbm.at[pl.ds(base, _C)], g_v, sem)
    c3 = pltpu.async_copy(ids_hbm.at[pl.ds(base, _C)], i_v, sem)
    c1.wait()
    c2.wait()
    c3.wait()

    first = i_v[pl.ds(0, _L)][0]
    last = i_v[pl.ds(_C - _L, _L)][_L - 1]
    lanes = lax.iota(jnp.int32, _L)

    def lane_sum(acc):
        half = [acc[2 * l] + acc[2 * l + 1] for l in range(_L // 2)]
        while len(half) > 1:
            half = [half[2 * l] + half[2 * l + 1]
                    for l in range(len(half) // 2)]
        return half[0]

    def seg_body(b, part):
        def abody(j, accs):
            new = []
            for k in range(4):
                sl = pl.ds((4 * j + k) * _L, _L)
                new.append(accs[k]
                           + jnp.where(i_v[sl] == b, s_v[sl] * g_v[sl], 0.0))
            return tuple(new)

        z = jnp.zeros((_L,), jnp.float32)
        a0, a1, a2, a3 = lax.fori_loop(0, _V // 4, abody, (z, z, z, z))
        return jnp.where(lanes == b, lane_sum((a0 + a1) + (a2 + a3)), part)

    part_v[...] = lax.fori_loop(first, last + 1, seg_body,
                                jnp.zeros((_L,), jnp.float32))

    pltpu.sync_copy(part_v, out_hbm.at[pl.ds(wid * _B, _B)])


def kernel(scores, gating, segment_ids):
    partials = _segsum_sc(scores, gating, segment_ids.astype(jnp.int32))
    return jnp.sum(partials.reshape(_NW, _B), axis=0)
